# baseline (speedup 1.0000x reference)
import jax
import jax.numpy as jnp
from jax import lax
from jax.experimental import pallas as pl
from jax.experimental.pallas import tpu as pltpu
import jax.experimental.pallas.tpu_sc as plsc

H = 768
E = 8
I = 1536
EPS = 1e-06
T = 2048
CAP = 256
SLOT = CAP + 16
NC = 2
NS = 16
L = 16
RPW = T // (NC * NS)
DEAD = T


def _rmsnorm(x, w, eps):
    v = jnp.mean(jnp.square(x), axis=-1, keepdims=True)
    return x * lax.rsqrt(v + eps) * w


def _quant_act(x):
    s = 127.0 / jnp.clip(jnp.max(jnp.abs(x), axis=-1, keepdims=True), 1e-5, None)
    return jnp.clip(jnp.round(x * s), -128.0, 127.0) / s


def _quant_w(w):
    s = 1.0 / jnp.clip(jnp.mean(jnp.abs(w)), 1e-5, None)
    return jnp.clip(jnp.round(w * s), -1.0, 1.0) / s


def _bitlinear(x, w, nw):
    xq = _quant_act(_rmsnorm(x, nw, 1e-8))
    wq = _quant_w(w)
    return jnp.dot(xq.astype(jnp.bfloat16), wq.astype(jnp.bfloat16),
                   preferred_element_type=jnp.float32)


RB = 128


def _router_body(x_ref, nw_ref, gw_ref, mask_ref, rank_ref, carry):
    x = x_ref[...]
    xn = _rmsnorm(x, nw_ref[...], EPS)
    logits = jnp.dot(xn.astype(jnp.bfloat16), gw_ref[...].astype(jnp.bfloat16),
                     preferred_element_type=jnp.float32)
    mx = jnp.max(logits, axis=-1, keepdims=True)
    p = jnp.exp(logits - mx)
    probs = p / jnp.sum(p, axis=-1, keepdims=True)
    ie = lax.broadcasted_iota(jnp.int32, logits.shape, 1)
    i1 = jnp.min(jnp.where(logits == mx, ie, E), axis=-1, keepdims=True)
    one = ie == i1
    logits2 = jnp.where(one, -jnp.inf, logits)
    l2 = jnp.max(logits2, axis=-1, keepdims=True)
    i2 = jnp.min(jnp.where(logits2 == l2, ie, E), axis=-1, keepdims=True)
    two = ie == i2
    m1 = jnp.max(jnp.where(one, probs, 0.0), axis=-1, keepdims=True)
    m2 = jnp.max(jnp.where(two, probs, 0.0), axis=-1, keepdims=True)
    mask_ref[...] = jnp.where(one, m1, 0.0) + jnp.where(two, -m2, 0.0)

    @pl.when(pl.program_id(0) == 0)
    def _():
        carry[...] = jnp.zeros((1, E), jnp.float32)

    hit = jnp.where(one | two, 1.0, 0.0)
    ir = lax.broadcasted_iota(jnp.int32, (RB, RB), 0)
    ic = lax.broadcasted_iota(jnp.int32, (RB, RB), 1)
    tril = jnp.where(ir > ic, 1.0, 0.0).astype(jnp.bfloat16)
    within = jnp.dot(tril, hit.astype(jnp.bfloat16),
                     preferred_element_type=jnp.float32)
    rank_ref[...] = (within + carry[...]).astype(jnp.int32)
    carry[...] = carry[...] + jnp.sum(hit, axis=0, keepdims=True)


def _router(xf, gate_norm_w, gate_w):
    return pl.pallas_call(
        _router_body,
        grid=(T // RB,),
        in_specs=[
            pl.BlockSpec((RB, H), lambda t: (t, 0)),
            pl.BlockSpec((1, H), lambda t: (0, 0)),
            pl.BlockSpec((H, E), lambda t: (0, 0)),
        ],
        out_specs=[
            pl.BlockSpec((RB, E), lambda t: (t, 0)),
            pl.BlockSpec((RB, E), lambda t: (t, 0)),
        ],
        out_shape=[
            jax.ShapeDtypeStruct((T, E), jnp.float32),
            jax.ShapeDtypeStruct((T, E), jnp.int32),
        ],
        scratch_shapes=[pltpu.VMEM((1, E), jnp.float32)],
    )(xf, gate_norm_w.reshape(1, H), gate_w)


def _dispatch_body(mask_hbm, rank_hbm, xf_hbm, sel_out, wsel_out, gidx_out,
                   xd_out, mask_v, rank_v, sel_v, wsel_v, gidx_v, sp_sel,
                   idx_v, rows_v, sem):
    c = lax.axis_index("c")
    s = lax.axis_index("s")

    @pl.when(s < 4)
    def _():
        e = c * 4 + s
        for i in range(SLOT // L):
            sel_v[pl.ds(i * L, L)] = jnp.zeros((L,), jnp.int32)
            wsel_v[pl.ds(i * L, L)] = jnp.zeros((L,), jnp.float32)

        def initg(i, _):
            gidx_v[pl.ds(i * L, L)] = jnp.full((L,), DEAD, jnp.int32)
            return 0

        lax.fori_loop(0, 2 * T // L, initg, 0)

        pltpu.sync_copy(mask_hbm, mask_v)
        pltpu.sync_copy(rank_hbm, rank_v)

        def step(t, _):
            loc = t * L + lax.iota(jnp.int32, L)
            col = plsc.load_gather(mask_v, [loc * E + e])
            r = plsc.load_gather(rank_v, [loc * E + e])
            m = col != 0.0
            kbit = jnp.where(col < 0.0, 1, 0).astype(jnp.int32)
            keep = m & (r < CAP)
            slot = jnp.where(keep, r, CAP)
            plsc.store_scatter(sel_v, [slot], loc, mask=keep)
            plsc.store_scatter(wsel_v, [slot], jnp.abs(col), mask=keep)
            plsc.store_scatter(gidx_v, [2 * loc + kbit],
                               e * CAP + slot, mask=keep)
            return 0

        lax.fori_loop(0, T // L, step, 0)
        pltpu.sync_copy(sel_v, sel_out.at[e])
        pltpu.sync_copy(wsel_v, wsel_out.at[e])
        pltpu.sync_copy(gidx_v, gidx_out.at[e])
        pltpu.sync_copy(sel_v, sp_sel.at[pl.ds(s * SLOT, SLOT)])

    plsc.subcore_barrier()

    e_loc = s // 4
    c0 = (s % 4) * RPW
    pltpu.sync_copy(sp_sel.at[pl.ds(e_loc * SLOT + c0, RPW)], idx_v)
    rbase = (c * 4 + e_loc) * CAP + c0
    CH = 32
    for k in range(RPW // CH):
        pltpu.async_copy(xf_hbm.at[idx_v.at[pl.ds(k * CH, CH)]],
                         rows_v, sem).wait()
        pltpu.sync_copy(rows_v, xd_out.at[pl.ds(rbase + k * CH, CH)])


def _dispatch(mask, rank, xf):
    mesh = plsc.VectorSubcoreMesh(core_axis_name="c", subcore_axis_name="s",
                                  num_cores=NC, num_subcores=NS)
    return pl.kernel(
        _dispatch_body,
        out_type=[
            jax.ShapeDtypeStruct((E, SLOT), jnp.int32),
            jax.ShapeDtypeStruct((E, SLOT), jnp.float32),
            jax.ShapeDtypeStruct((E, 2 * T), jnp.int32),
            jax.ShapeDtypeStruct((T, H), jnp.float32),
        ],
        mesh=mesh,
        compiler_params=pltpu.CompilerParams(needs_layout_passes=False),
        scratch_types=[
            pltpu.VMEM((T * E,), jnp.float32),
            pltpu.VMEM((T * E,), jnp.int32),
            pltpu.VMEM((SLOT,), jnp.int32),
            pltpu.VMEM((SLOT,), jnp.float32),
            pltpu.VMEM((2 * T,), jnp.int32),
            pltpu.VMEM_SHARED((4 * SLOT,), jnp.int32),
            pltpu.VMEM((RPW,), jnp.int32),
            pltpu.VMEM((32, H), jnp.float32),
            pltpu.SemaphoreType.DMA,
        ],
    )(mask, rank, xf)


def _expert_body(xd_ref, gw_ref, gnw_ref, dw_ref, dnw_ref, wsel_ref, out_ref):
    x = xd_ref[...]
    y = _bitlinear(x, gw_ref[0], gnw_ref[0])
    gate = y[:, :I]
    up = y[:, I:]
    z = gate * jax.nn.sigmoid(gate) * up
    o = _bitlinear(z, dw_ref[0], dnw_ref[0])
    out_ref[...] = o * wsel_ref[0]


def _experts(xd, gw, gnw, dw, dnw, wsel):
    return pl.pallas_call(
        _expert_body,
        grid=(E,),
        in_specs=[
            pl.BlockSpec((CAP, H), lambda e: (e, 0)),
            pl.BlockSpec((1, H, 2 * I), lambda e: (e, 0, 0)),
            pl.BlockSpec((1, 1, H), lambda e: (e, 0, 0)),
            pl.BlockSpec((1, I, H), lambda e: (e, 0, 0)),
            pl.BlockSpec((1, 1, I), lambda e: (e, 0, 0)),
            pl.BlockSpec((1, CAP, 1), lambda e: (e, 0, 0)),
        ],
        out_specs=pl.BlockSpec((CAP, H), lambda e: (e, 0)),
        out_shape=jax.ShapeDtypeStruct((T, H), jnp.float32),
    )(xd, gw, gnw.reshape(E, 1, H), dw, dnw.reshape(E, 1, I), wsel)


def _gather_body(eo_hbm, gidxt_hbm, g_out, idx8_v, idx_v, rows_v, sem):
    c = lax.axis_index("c")
    s = lax.axis_index("s")
    w = c * NS + s
    t0 = w * RPW

    pltpu.sync_copy(gidxt_hbm.at[pl.ds(2 * t0 * E, 2 * RPW * E)], idx8_v)

    for cc in range(2 * RPW // L):
        base = cc * L + lax.iota(jnp.int32, L)
        m = plsc.load_gather(idx8_v, [base * E])
        for j in range(1, E):
            m = jnp.minimum(m, plsc.load_gather(idx8_v, [base * E + j]))
        idx_v[pl.ds(cc * L, L)] = m

    CH = 32
    for k in range(2 * RPW // CH):
        pltpu.async_copy(eo_hbm.at[idx_v.at[pl.ds(k * CH, CH)]],
                         rows_v, sem).wait()
        pltpu.sync_copy(rows_v, g_out.at[pl.ds(2 * t0 + k * CH, CH)])


def _gather(eo_ext, gidxt):
    mesh = plsc.VectorSubcoreMesh(core_axis_name="c", subcore_axis_name="s",
                                  num_cores=NC, num_subcores=NS)
    return pl.kernel(
        _gather_body,
        out_type=jax.ShapeDtypeStruct((2 * T, H), jnp.float32),
        mesh=mesh,
        compiler_params=pltpu.CompilerParams(needs_layout_passes=False),
        scratch_types=[
            pltpu.VMEM((2 * RPW * E,), jnp.int32),
            pltpu.VMEM((2 * RPW,), jnp.int32),
            pltpu.VMEM((32, H), jnp.float32),
            pltpu.SemaphoreType.DMA,
        ],
    )(eo_ext, gidxt)


def _shared_body(x_ref, gw_ref, gnw_ref, dw_ref, dnw_ref, sgw_ref,
                 g_ref, out_ref):
    x = x_ref[...]
    y = _bitlinear(x, gw_ref[...], gnw_ref[...])
    gate = y[:, :I]
    up = y[:, I:]
    z = gate * jax.nn.sigmoid(gate) * up
    o = _bitlinear(z, dw_ref[...], dnw_ref[...])
    score = jax.nn.sigmoid(jnp.sum(x * sgw_ref[...], axis=-1, keepdims=True))
    out_ref[...] = o * score + g_ref[:, 0, :] + g_ref[:, 1, :]


def _shared_combine(xf, gw, gnw, dw, dnw, sgw, g):
    TB = 256
    return pl.pallas_call(
        _shared_body,
        grid=(T // TB,),
        in_specs=[
            pl.BlockSpec((TB, H), lambda t: (t, 0)),
            pl.BlockSpec((H, 2 * I), lambda t: (0, 0)),
            pl.BlockSpec((1, H), lambda t: (0, 0)),
            pl.BlockSpec((I, H), lambda t: (0, 0)),
            pl.BlockSpec((1, I), lambda t: (0, 0)),
            pl.BlockSpec((1, H), lambda t: (0, 0)),
            pl.BlockSpec((TB, 2, H), lambda t: (t, 0, 0)),
        ],
        out_specs=pl.BlockSpec((TB, H), lambda t: (t, 0)),
        out_shape=jax.ShapeDtypeStruct((T, H), jnp.float32),
    )(xf, gw, gnw.reshape(1, H), dw, dnw.reshape(1, I),
      sgw.reshape(1, H), g)


@jax.jit
def _impl(x, expert_gate_w, expert_gate_nw, expert_down_w, expert_down_nw,
          shared_gate_w, shared_gate_nw, shared_down_w, shared_down_nw,
          gate_norm_w, gate_w, shared_expert_gate_w):
    Bm, Sm, Hm = x.shape
    xf = x.reshape(T, Hm)
    mask, rank = _router(xf, gate_norm_w, gate_w)
    sel, wsel, gidx8, xd = _dispatch(mask.reshape(T * E),
                                     rank.reshape(T * E), xf)
    wsel3 = wsel[:, :CAP].reshape(E, CAP, 1)
    eo = _experts(xd, expert_gate_w, expert_gate_nw,
                  expert_down_w, expert_down_nw, wsel3)
    eo_ext = jnp.concatenate([eo, jnp.zeros((L, H), jnp.float32)], axis=0)
    gidxt = gidx8.T.reshape(2 * T * E)
    g = _gather(eo_ext, gidxt)
    final = _shared_combine(xf, shared_gate_w, shared_gate_nw,
                            shared_down_w, shared_down_nw,
                            shared_expert_gate_w, g.reshape(T, 2, H))
    return final.reshape(Bm, Sm, Hm)


def kernel(x, expert_gate_w, expert_gate_nw, expert_down_w, expert_down_nw,
           shared_gate_w, shared_gate_nw, shared_down_w, shared_down_nw,
           gate_norm_w, gate_w, shared_expert_gate_w):
    return _impl(x, expert_gate_w, expert_gate_nw, expert_down_w,
                 expert_down_nw, shared_gate_w, shared_gate_nw,
                 shared_down_w, shared_down_nw, gate_norm_w, gate_w,
                 shared_expert_gate_w)


# speedup vs baseline: 1.0082x; 1.0082x over previous
import jax
import jax.numpy as jnp
from jax import lax
from jax.experimental import pallas as pl
from jax.experimental.pallas import tpu as pltpu
import jax.experimental.pallas.tpu_sc as plsc

H = 768
E = 8
I = 1536
EPS = 1e-06
T = 2048
CAP = 256
SLOT = CAP + 16
NC = 2
NS = 16
L = 16
RPW = T // (NC * NS)
DEAD = T


def _rmsnorm(x, w, eps):
    v = jnp.mean(jnp.square(x), axis=-1, keepdims=True)
    return x * lax.rsqrt(v + eps) * w


def _quant_act(x):
    s = 127.0 / jnp.clip(jnp.max(jnp.abs(x), axis=-1, keepdims=True), 1e-5, None)
    return jnp.clip(jnp.round(x * s), -128.0, 127.0) / s


def _quant_w(w):
    s = 1.0 / jnp.clip(jnp.mean(jnp.abs(w)), 1e-5, None)
    return jnp.clip(jnp.round(w * s), -1.0, 1.0) / s


def _bitlinear(x, w, nw):
    xq = _quant_act(_rmsnorm(x, nw, 1e-8))
    wq = _quant_w(w)
    return jnp.dot(xq.astype(jnp.bfloat16), wq.astype(jnp.bfloat16),
                   preferred_element_type=jnp.float32)


RB = 128


def _router_body(x_ref, nw_ref, gw_ref, mask_ref, rank_ref, carry):
    x = x_ref[...]
    xn = _rmsnorm(x, nw_ref[...], EPS)
    logits = jnp.dot(xn.astype(jnp.bfloat16), gw_ref[...].astype(jnp.bfloat16),
                     preferred_element_type=jnp.float32)
    mx = jnp.max(logits, axis=-1, keepdims=True)
    p = jnp.exp(logits - mx)
    probs = p / jnp.sum(p, axis=-1, keepdims=True)
    ie = lax.broadcasted_iota(jnp.int32, logits.shape, 1)
    i1 = jnp.min(jnp.where(logits == mx, ie, E), axis=-1, keepdims=True)
    one = ie == i1
    logits2 = jnp.where(one, -jnp.inf, logits)
    l2 = jnp.max(logits2, axis=-1, keepdims=True)
    i2 = jnp.min(jnp.where(logits2 == l2, ie, E), axis=-1, keepdims=True)
    two = ie == i2
    m1 = jnp.max(jnp.where(one, probs, 0.0), axis=-1, keepdims=True)
    m2 = jnp.max(jnp.where(two, probs, 0.0), axis=-1, keepdims=True)
    mask_ref[...] = jnp.where(one, m1, 0.0) + jnp.where(two, -m2, 0.0)

    @pl.when(pl.program_id(0) == 0)
    def _():
        carry[...] = jnp.zeros((1, E), jnp.float32)

    hit = jnp.where(one | two, 1.0, 0.0)
    ir = lax.broadcasted_iota(jnp.int32, (RB, RB), 0)
    ic = lax.broadcasted_iota(jnp.int32, (RB, RB), 1)
    tril = jnp.where(ir > ic, 1.0, 0.0).astype(jnp.bfloat16)
    within = jnp.dot(tril, hit.astype(jnp.bfloat16),
                     preferred_element_type=jnp.float32)
    rank_ref[...] = (within + carry[...]).astype(jnp.int32)
    carry[...] = carry[...] + jnp.sum(hit, axis=0, keepdims=True)


def _router(xf, gate_norm_w, gate_w):
    return pl.pallas_call(
        _router_body,
        grid=(T // RB,),
        in_specs=[
            pl.BlockSpec((RB, H), lambda t: (t, 0)),
            pl.BlockSpec((1, H), lambda t: (0, 0)),
            pl.BlockSpec((H, E), lambda t: (0, 0)),
        ],
        out_specs=[
            pl.BlockSpec((RB, E), lambda t: (t, 0)),
            pl.BlockSpec((RB, E), lambda t: (t, 0)),
        ],
        out_shape=[
            jax.ShapeDtypeStruct((T, E), jnp.float32),
            jax.ShapeDtypeStruct((T, E), jnp.int32),
        ],
        scratch_shapes=[pltpu.VMEM((1, E), jnp.float32)],
    )(xf, gate_norm_w.reshape(1, H), gate_w)


def _dispatch_body(mask_hbm, rank_hbm, xf_hbm, sel_out, wsel_out, gidx_out,
                   xd_out, mask_v, rank_v, sel_v, wsel_v, gidx_v, sp_sel,
                   idx_v, rows_v, sem):
    c = lax.axis_index("c")
    s = lax.axis_index("s")

    @pl.when(s < 4)
    def _():
        e = c * 4 + s
        for i in range(SLOT // L):
            sel_v[pl.ds(i * L, L)] = jnp.zeros((L,), jnp.int32)
            wsel_v[pl.ds(i * L, L)] = jnp.zeros((L,), jnp.float32)

        def initg(i, _):
            gidx_v[pl.ds(i * L, L)] = jnp.full((L,), DEAD, jnp.int32)
            return 0

        lax.fori_loop(0, 2 * T // L, initg, 0)

        pltpu.sync_copy(mask_hbm, mask_v)
        pltpu.sync_copy(rank_hbm, rank_v)

        def step(t, _):
            loc = t * L + lax.iota(jnp.int32, L)
            col = plsc.load_gather(mask_v, [loc * E + e])
            r = plsc.load_gather(rank_v, [loc * E + e])
            m = col != 0.0
            kbit = jnp.where(col < 0.0, 1, 0).astype(jnp.int32)
            keep = m & (r < CAP)
            slot = jnp.where(keep, r, CAP)
            plsc.store_scatter(sel_v, [slot], loc, mask=keep)
            plsc.store_scatter(wsel_v, [slot], jnp.abs(col), mask=keep)
            plsc.store_scatter(gidx_v, [2 * loc + kbit],
                               e * CAP + slot, mask=keep)
            return 0

        lax.fori_loop(0, T // L, step, 0)
        pltpu.sync_copy(sel_v, sel_out.at[e])
        pltpu.sync_copy(wsel_v, wsel_out.at[e])
        pltpu.sync_copy(gidx_v, gidx_out.at[e])
        pltpu.sync_copy(sel_v, sp_sel.at[pl.ds(s * SLOT, SLOT)])

    plsc.subcore_barrier()

    e_loc = s // 4
    c0 = (s % 4) * RPW
    pltpu.sync_copy(sp_sel.at[pl.ds(e_loc * SLOT + c0, RPW)], idx_v)
    rbase = (c * 4 + e_loc) * CAP + c0
    pltpu.async_copy(xf_hbm.at[idx_v], rows_v, sem).wait()
    pltpu.sync_copy(rows_v, xd_out.at[pl.ds(rbase, RPW)])


def _dispatch(mask, rank, xf):
    mesh = plsc.VectorSubcoreMesh(core_axis_name="c", subcore_axis_name="s",
                                  num_cores=NC, num_subcores=NS)
    return pl.kernel(
        _dispatch_body,
        out_type=[
            jax.ShapeDtypeStruct((E, SLOT), jnp.int32),
            jax.ShapeDtypeStruct((E, SLOT), jnp.float32),
            jax.ShapeDtypeStruct((E, 2 * T), jnp.int32),
            jax.ShapeDtypeStruct((T, H), jnp.float32),
        ],
        mesh=mesh,
        compiler_params=pltpu.CompilerParams(needs_layout_passes=False),
        scratch_types=[
            pltpu.VMEM((T * E,), jnp.float32),
            pltpu.VMEM((T * E,), jnp.int32),
            pltpu.VMEM((SLOT,), jnp.int32),
            pltpu.VMEM((SLOT,), jnp.float32),
            pltpu.VMEM((2 * T,), jnp.int32),
            pltpu.VMEM_SHARED((4 * SLOT,), jnp.int32),
            pltpu.VMEM((RPW,), jnp.int32),
            pltpu.VMEM((RPW, H), jnp.float32),
            pltpu.SemaphoreType.DMA,
        ],
    )(mask, rank, xf)


def _expert_body(xd_ref, gw_ref, gnw_ref, dw_ref, dnw_ref, wsel_ref, out_ref):
    x = xd_ref[...]
    y = _bitlinear(x, gw_ref[0], gnw_ref[0])
    gate = y[:, :I]
    up = y[:, I:]
    z = gate * jax.nn.sigmoid(gate) * up
    o = _bitlinear(z, dw_ref[0], dnw_ref[0])
    out_ref[...] = o * wsel_ref[0]


def _experts(xd, gw, gnw, dw, dnw, wsel):
    return pl.pallas_call(
        _expert_body,
        grid=(E,),
        in_specs=[
            pl.BlockSpec((CAP, H), lambda e: (e, 0)),
            pl.BlockSpec((1, H, 2 * I), lambda e: (e, 0, 0)),
            pl.BlockSpec((1, 1, H), lambda e: (e, 0, 0)),
            pl.BlockSpec((1, I, H), lambda e: (e, 0, 0)),
            pl.BlockSpec((1, 1, I), lambda e: (e, 0, 0)),
            pl.BlockSpec((1, CAP, 1), lambda e: (e, 0, 0)),
        ],
        out_specs=pl.BlockSpec((CAP, H), lambda e: (e, 0)),
        out_shape=jax.ShapeDtypeStruct((T, H), jnp.float32),
    )(xd, gw, gnw.reshape(E, 1, H), dw, dnw.reshape(E, 1, I), wsel)


def _gather_body(eo_hbm, gidxt_hbm, g_out, idx8_v, idx_v, rows_a, rows_b,
                 sem_a, sem_b):
    c = lax.axis_index("c")
    s = lax.axis_index("s")
    w = c * NS + s
    t0 = w * RPW

    pltpu.sync_copy(gidxt_hbm.at[pl.ds(2 * t0 * E, 2 * RPW * E)], idx8_v)

    for cc in range(2 * RPW // L):
        base = cc * L + lax.iota(jnp.int32, L)
        m = plsc.load_gather(idx8_v, [base * E])
        for j in range(1, E):
            m = jnp.minimum(m, plsc.load_gather(idx8_v, [base * E + j]))
        idx_v[pl.ds(cc * L, L)] = m

    cp_a = pltpu.async_copy(eo_hbm.at[idx_v.at[pl.ds(0, RPW)]], rows_a, sem_a)
    cp_b = pltpu.async_copy(eo_hbm.at[idx_v.at[pl.ds(RPW, RPW)]], rows_b, sem_b)
    cp_a.wait()
    pltpu.sync_copy(rows_a, g_out.at[pl.ds(2 * t0, RPW)])
    cp_b.wait()
    pltpu.sync_copy(rows_b, g_out.at[pl.ds(2 * t0 + RPW, RPW)])


def _gather(eo_ext, gidxt):
    mesh = plsc.VectorSubcoreMesh(core_axis_name="c", subcore_axis_name="s",
                                  num_cores=NC, num_subcores=NS)
    return pl.kernel(
        _gather_body,
        out_type=jax.ShapeDtypeStruct((2 * T, H), jnp.float32),
        mesh=mesh,
        compiler_params=pltpu.CompilerParams(needs_layout_passes=False),
        scratch_types=[
            pltpu.VMEM((2 * RPW * E,), jnp.int32),
            pltpu.VMEM((2 * RPW,), jnp.int32),
            pltpu.VMEM((RPW, H), jnp.float32),
            pltpu.VMEM((RPW, H), jnp.float32),
            pltpu.SemaphoreType.DMA,
            pltpu.SemaphoreType.DMA,
        ],
    )(eo_ext, gidxt)


def _shared_body(x_ref, gw_ref, gnw_ref, dw_ref, dnw_ref, sgw_ref,
                 g_ref, out_ref):
    x = x_ref[...]
    y = _bitlinear(x, gw_ref[...], gnw_ref[...])
    gate = y[:, :I]
    up = y[:, I:]
    z = gate * jax.nn.sigmoid(gate) * up
    o = _bitlinear(z, dw_ref[...], dnw_ref[...])
    score = jax.nn.sigmoid(jnp.sum(x * sgw_ref[...], axis=-1, keepdims=True))
    out_ref[...] = o * score + g_ref[:, 0, :] + g_ref[:, 1, :]


def _shared_combine(xf, gw, gnw, dw, dnw, sgw, g):
    TB = 256
    return pl.pallas_call(
        _shared_body,
        grid=(T // TB,),
        in_specs=[
            pl.BlockSpec((TB, H), lambda t: (t, 0)),
            pl.BlockSpec((H, 2 * I), lambda t: (0, 0)),
            pl.BlockSpec((1, H), lambda t: (0, 0)),
            pl.BlockSpec((I, H), lambda t: (0, 0)),
            pl.BlockSpec((1, I), lambda t: (0, 0)),
            pl.BlockSpec((1, H), lambda t: (0, 0)),
            pl.BlockSpec((TB, 2, H), lambda t: (t, 0, 0)),
        ],
        out_specs=pl.BlockSpec((TB, H), lambda t: (t, 0)),
        out_shape=jax.ShapeDtypeStruct((T, H), jnp.float32),
    )(xf, gw, gnw.reshape(1, H), dw, dnw.reshape(1, I),
      sgw.reshape(1, H), g)


@jax.jit
def _impl(x, expert_gate_w, expert_gate_nw, expert_down_w, expert_down_nw,
          shared_gate_w, shared_gate_nw, shared_down_w, shared_down_nw,
          gate_norm_w, gate_w, shared_expert_gate_w):
    Bm, Sm, Hm = x.shape
    xf = x.reshape(T, Hm)
    mask, rank = _router(xf, gate_norm_w, gate_w)
    sel, wsel, gidx8, xd = _dispatch(mask.reshape(T * E),
                                     rank.reshape(T * E), xf)
    wsel3 = wsel[:, :CAP].reshape(E, CAP, 1)
    eo = _experts(xd, expert_gate_w, expert_gate_nw,
                  expert_down_w, expert_down_nw, wsel3)
    eo_ext = jnp.concatenate([eo, jnp.zeros((L, H), jnp.float32)], axis=0)
    gidxt = gidx8.T.reshape(2 * T * E)
    g = _gather(eo_ext, gidxt)
    final = _shared_combine(xf, shared_gate_w, shared_gate_nw,
                            shared_down_w, shared_down_nw,
                            shared_expert_gate_w, g.reshape(T, 2, H))
    return final.reshape(Bm, Sm, Hm)


def kernel(x, expert_gate_w, expert_gate_nw, expert_down_w, expert_down_nw,
           shared_gate_w, shared_gate_nw, shared_down_w, shared_down_nw,
           gate_norm_w, gate_w, shared_expert_gate_w):
    return _impl(x, expert_gate_w, expert_gate_nw, expert_down_w,
                 expert_down_nw, shared_gate_w, shared_gate_nw,
                 shared_down_w, shared_down_nw, gate_norm_w, gate_w,
                 shared_expert_gate_w)


# speedup vs baseline: 1.1792x; 1.1697x over previous
import jax
import jax.numpy as jnp
from jax import lax
from jax.experimental import pallas as pl
from jax.experimental.pallas import tpu as pltpu
import jax.experimental.pallas.tpu_sc as plsc

H = 768
E = 8
I = 1536
EPS = 1e-06
T = 2048
CAP = 256
SLOT = CAP + 16
NC = 2
NS = 16
L = 16
RPW = T // (NC * NS)
DEAD = T


def _rmsnorm(x, w, eps):
    v = jnp.mean(jnp.square(x), axis=-1, keepdims=True)
    return x * lax.rsqrt(v + eps) * w


def _quant_act(x):
    s = 127.0 / jnp.clip(jnp.max(jnp.abs(x), axis=-1, keepdims=True), 1e-5, None)
    return jnp.clip(jnp.round(x * s), -128.0, 127.0) / s


def _quant_w(w):
    s = 1.0 / jnp.clip(jnp.mean(jnp.abs(w)), 1e-5, None)
    return jnp.clip(jnp.round(w * s), -1.0, 1.0) / s


def _bitlinear(x, w, nw):
    xq = _quant_act(_rmsnorm(x, nw, 1e-8))
    wq = _quant_w(w)
    return jnp.dot(xq.astype(jnp.bfloat16), wq.astype(jnp.bfloat16),
                   preferred_element_type=jnp.float32)


RB = 128


def _router_body(x_ref, nw_ref, gw_ref, mask_ref, rank_ref, carry):
    x = x_ref[...]
    xn = _rmsnorm(x, nw_ref[...], EPS)
    logits = jnp.dot(xn.astype(jnp.bfloat16), gw_ref[...].astype(jnp.bfloat16),
                     preferred_element_type=jnp.float32)
    mx = jnp.max(logits, axis=-1, keepdims=True)
    p = jnp.exp(logits - mx)
    probs = p / jnp.sum(p, axis=-1, keepdims=True)
    ie = lax.broadcasted_iota(jnp.int32, logits.shape, 1)
    i1 = jnp.min(jnp.where(logits == mx, ie, E), axis=-1, keepdims=True)
    one = ie == i1
    logits2 = jnp.where(one, -jnp.inf, logits)
    l2 = jnp.max(logits2, axis=-1, keepdims=True)
    i2 = jnp.min(jnp.where(logits2 == l2, ie, E), axis=-1, keepdims=True)
    two = ie == i2
    m1 = jnp.max(jnp.where(one, probs, 0.0), axis=-1, keepdims=True)
    m2 = jnp.max(jnp.where(two, probs, 0.0), axis=-1, keepdims=True)
    mask_ref[...] = jnp.where(one, m1, 0.0) + jnp.where(two, -m2, 0.0)

    @pl.when(pl.program_id(0) == 0)
    def _():
        carry[...] = jnp.zeros((1, E), jnp.float32)

    hit = jnp.where(one | two, 1.0, 0.0)
    ir = lax.broadcasted_iota(jnp.int32, (RB, RB), 0)
    ic = lax.broadcasted_iota(jnp.int32, (RB, RB), 1)
    tril = jnp.where(ir > ic, 1.0, 0.0).astype(jnp.bfloat16)
    within = jnp.dot(tril, hit.astype(jnp.bfloat16),
                     preferred_element_type=jnp.float32)
    rank_ref[...] = (within + carry[...]).astype(jnp.int32)
    carry[...] = carry[...] + jnp.sum(hit, axis=0, keepdims=True)


def _router(xf, gate_norm_w, gate_w):
    return pl.pallas_call(
        _router_body,
        grid=(T // RB,),
        in_specs=[
            pl.BlockSpec((RB, H), lambda t: (t, 0)),
            pl.BlockSpec((1, H), lambda t: (0, 0)),
            pl.BlockSpec((H, E), lambda t: (0, 0)),
        ],
        out_specs=[
            pl.BlockSpec((RB, E), lambda t: (t, 0)),
            pl.BlockSpec((RB, E), lambda t: (t, 0)),
        ],
        out_shape=[
            jax.ShapeDtypeStruct((T, E), jnp.float32),
            jax.ShapeDtypeStruct((T, E), jnp.int32),
        ],
        scratch_shapes=[pltpu.VMEM((1, E), jnp.float32)],
    )(xf, gate_norm_w.reshape(1, H), gate_w)


def _dispatch_body(mask_hbm, rank_hbm, xf_hbm, sel_out, wsel_out, gidx_out,
                   xd_out, mask_v, rank_v, sel_v, wsel_v, gidx_v, sp_sel,
                   idx_v, rows_v, sem):
    c = lax.axis_index("c")
    s = lax.axis_index("s")

    @pl.when(s < 4)
    def _():
        e = c * 4 + s
        for i in range(SLOT // L):
            sel_v[pl.ds(i * L, L)] = jnp.zeros((L,), jnp.int32)
            wsel_v[pl.ds(i * L, L)] = jnp.zeros((L,), jnp.float32)

        def initg(i, _):
            gidx_v[pl.ds(i * L, L)] = jnp.full((L,), DEAD, jnp.int32)
            return 0

        lax.fori_loop(0, 2 * T // L, initg, 0)

        pltpu.sync_copy(mask_hbm, mask_v)
        pltpu.sync_copy(rank_hbm, rank_v)

        def step(t, _):
            loc = t * L + lax.iota(jnp.int32, L)
            col = plsc.load_gather(mask_v, [loc * E + e])
            r = plsc.load_gather(rank_v, [loc * E + e])
            m = col != 0.0
            kbit = jnp.where(col < 0.0, 1, 0).astype(jnp.int32)
            keep = m & (r < CAP)
            slot = jnp.where(keep, r, CAP)
            plsc.store_scatter(sel_v, [slot], loc, mask=keep)
            plsc.store_scatter(wsel_v, [slot], jnp.abs(col), mask=keep)
            plsc.store_scatter(gidx_v, [2 * loc + kbit],
                               e * CAP + slot, mask=keep)
            return 0

        lax.fori_loop(0, T // L, step, 0)
        pltpu.sync_copy(sel_v, sel_out.at[e])
        pltpu.sync_copy(wsel_v, wsel_out.at[e])
        pltpu.sync_copy(gidx_v, gidx_out.at[e])
        pltpu.sync_copy(sel_v, sp_sel.at[pl.ds(s * SLOT, SLOT)])

    plsc.subcore_barrier()

    e_loc = s // 4
    c0 = (s % 4) * RPW
    pltpu.sync_copy(sp_sel.at[pl.ds(e_loc * SLOT + c0, RPW)], idx_v)
    rbase = (c * 4 + e_loc) * CAP + c0
    pltpu.async_copy(xf_hbm.at[idx_v], rows_v, sem).wait()
    pltpu.sync_copy(rows_v, xd_out.at[pl.ds(rbase, RPW)])


def _dispatch(mask, rank, xf):
    mesh = plsc.VectorSubcoreMesh(core_axis_name="c", subcore_axis_name="s",
                                  num_cores=NC, num_subcores=NS)
    return pl.kernel(
        _dispatch_body,
        out_type=[
            jax.ShapeDtypeStruct((E, SLOT), jnp.int32),
            jax.ShapeDtypeStruct((E, SLOT), jnp.float32),
            jax.ShapeDtypeStruct((E, 2 * T), jnp.int32),
            jax.ShapeDtypeStruct((T, H), jnp.float32),
        ],
        mesh=mesh,
        compiler_params=pltpu.CompilerParams(needs_layout_passes=False),
        scratch_types=[
            pltpu.VMEM((T * E,), jnp.float32),
            pltpu.VMEM((T * E,), jnp.int32),
            pltpu.VMEM((SLOT,), jnp.int32),
            pltpu.VMEM((SLOT,), jnp.float32),
            pltpu.VMEM((2 * T,), jnp.int32),
            pltpu.VMEM_SHARED((4 * SLOT,), jnp.int32),
            pltpu.VMEM((RPW,), jnp.int32),
            pltpu.VMEM((RPW, H), jnp.float32),
            pltpu.SemaphoreType.DMA,
        ],
    )(mask, rank, xf)


def _expert_body(xd_ref, gw_ref, gnw_ref, dw_ref, dnw_ref, wsel_ref, out_ref):
    x = xd_ref[...]
    y = _bitlinear(x, gw_ref[0], gnw_ref[0])
    gate = y[:, :I]
    up = y[:, I:]
    z = gate * jax.nn.sigmoid(gate) * up
    o = _bitlinear(z, dw_ref[0], dnw_ref[0])
    out_ref[...] = o * wsel_ref[0]


def _experts(xd, gw, gnw, dw, dnw, wsel):
    return pl.pallas_call(
        _expert_body,
        grid=(E,),
        in_specs=[
            pl.BlockSpec((CAP, H), lambda e: (e, 0)),
            pl.BlockSpec((1, H, 2 * I), lambda e: (e, 0, 0)),
            pl.BlockSpec((1, 1, H), lambda e: (e, 0, 0)),
            pl.BlockSpec((1, I, H), lambda e: (e, 0, 0)),
            pl.BlockSpec((1, 1, I), lambda e: (e, 0, 0)),
            pl.BlockSpec((1, CAP, 1), lambda e: (e, 0, 0)),
        ],
        out_specs=pl.BlockSpec((CAP, H), lambda e: (e, 0)),
        out_shape=jax.ShapeDtypeStruct((T, H), jnp.float32),
    )(xd, gw, gnw.reshape(E, 1, H), dw, dnw.reshape(E, 1, I), wsel)


def _gather_body(eo_hbm, gidxt_hbm, g_out, idx8_v, idx_v, rows_a, rows_b,
                 sem_a, sem_b):
    c = lax.axis_index("c")
    s = lax.axis_index("s")
    w = c * NS + s
    t0 = w * RPW

    pltpu.sync_copy(gidxt_hbm.at[pl.ds(2 * t0 * E, 2 * RPW * E)], idx8_v)

    for cc in range(2 * RPW // L):
        base = cc * L + lax.iota(jnp.int32, L)
        m = plsc.load_gather(idx8_v, [base * E])
        for j in range(1, E):
            m = jnp.minimum(m, plsc.load_gather(idx8_v, [base * E + j]))
        idx_v[pl.ds(cc * L, L)] = m

    cp_a = pltpu.async_copy(eo_hbm.at[idx_v.at[pl.ds(0, RPW)]], rows_a, sem_a)
    cp_b = pltpu.async_copy(eo_hbm.at[idx_v.at[pl.ds(RPW, RPW)]], rows_b, sem_b)
    cp_a.wait()
    pltpu.sync_copy(rows_a, g_out.at[pl.ds(2 * t0, RPW)])
    cp_b.wait()
    pltpu.sync_copy(rows_b, g_out.at[pl.ds(2 * t0 + RPW, RPW)])


def _gather(eo_ext, gidxt):
    mesh = plsc.VectorSubcoreMesh(core_axis_name="c", subcore_axis_name="s",
                                  num_cores=NC, num_subcores=NS)
    return pl.kernel(
        _gather_body,
        out_type=jax.ShapeDtypeStruct((2 * T, H), jnp.float32),
        mesh=mesh,
        compiler_params=pltpu.CompilerParams(needs_layout_passes=False),
        scratch_types=[
            pltpu.VMEM((2 * RPW * E,), jnp.int32),
            pltpu.VMEM((2 * RPW,), jnp.int32),
            pltpu.VMEM((RPW, H), jnp.float32),
            pltpu.VMEM((RPW, H), jnp.float32),
            pltpu.SemaphoreType.DMA,
            pltpu.SemaphoreType.DMA,
        ],
    )(eo_ext, gidxt)


def _shared_body(x_ref, gw_ref, gnw_ref, dw_ref, dnw_ref, sgw_ref, out_ref):
    x = x_ref[...]
    y = _bitlinear(x, gw_ref[...], gnw_ref[...])
    gate = y[:, :I]
    up = y[:, I:]
    z = gate * jax.nn.sigmoid(gate) * up
    o = _bitlinear(z, dw_ref[...], dnw_ref[...])
    score = jax.nn.sigmoid(jnp.sum(x * sgw_ref[...], axis=-1, keepdims=True))
    out_ref[...] = o * score


def _shared(xf, gw, gnw, dw, dnw, sgw):
    TB = 256
    return pl.pallas_call(
        _shared_body,
        grid=(T // TB,),
        in_specs=[
            pl.BlockSpec((TB, H), lambda t: (t, 0)),
            pl.BlockSpec((H, 2 * I), lambda t: (0, 0)),
            pl.BlockSpec((1, H), lambda t: (0, 0)),
            pl.BlockSpec((I, H), lambda t: (0, 0)),
            pl.BlockSpec((1, I), lambda t: (0, 0)),
            pl.BlockSpec((1, H), lambda t: (0, 0)),
        ],
        out_specs=pl.BlockSpec((TB, H), lambda t: (t, 0)),
        out_shape=jax.ShapeDtypeStruct((T, H), jnp.float32),
    )(xf, gw, gnw.reshape(1, H), dw, dnw.reshape(1, I), sgw.reshape(1, H))


def _final_body(sh_ref, g_ref, out_ref):
    out_ref[...] = sh_ref[...] + g_ref[:, 0, :] + g_ref[:, 1, :]


def _final_add(sh, g):
    TB = 256
    return pl.pallas_call(
        _final_body,
        grid=(T // TB,),
        in_specs=[
            pl.BlockSpec((TB, H), lambda t: (t, 0)),
            pl.BlockSpec((TB, 2, H), lambda t: (t, 0, 0)),
        ],
        out_specs=pl.BlockSpec((TB, H), lambda t: (t, 0)),
        out_shape=jax.ShapeDtypeStruct((T, H), jnp.float32),
    )(sh, g)


@jax.jit
def _impl(x, expert_gate_w, expert_gate_nw, expert_down_w, expert_down_nw,
          shared_gate_w, shared_gate_nw, shared_down_w, shared_down_nw,
          gate_norm_w, gate_w, shared_expert_gate_w):
    Bm, Sm, Hm = x.shape
    xf = x.reshape(T, Hm)
    mask, rank = _router(xf, gate_norm_w, gate_w)
    sel, wsel, gidx8, xd = _dispatch(mask.reshape(T * E),
                                     rank.reshape(T * E), xf)
    wsel3 = wsel[:, :CAP].reshape(E, CAP, 1)
    eo = _experts(xd, expert_gate_w, expert_gate_nw,
                  expert_down_w, expert_down_nw, wsel3)
    eo_ext = jnp.concatenate([eo, jnp.zeros((L, H), jnp.float32)], axis=0)
    gidxt = gidx8.T.reshape(2 * T * E)
    g = _gather(eo_ext, gidxt)
    sh = _shared(xf, shared_gate_w, shared_gate_nw,
                 shared_down_w, shared_down_nw, shared_expert_gate_w)
    final = _final_add(sh, g.reshape(T, 2, H))
    return final.reshape(Bm, Sm, Hm)


def kernel(x, expert_gate_w, expert_gate_nw, expert_down_w, expert_down_nw,
           shared_gate_w, shared_gate_nw, shared_down_w, shared_down_nw,
           gate_norm_w, gate_w, shared_expert_gate_w):
    return _impl(x, expert_gate_w, expert_gate_nw, expert_down_w,
                 expert_down_nw, shared_gate_w, shared_gate_nw,
                 shared_down_w, shared_down_nw, gate_norm_w, gate_w,
                 shared_expert_gate_w)
